# baseline (speedup 1.0000x reference)
import jax
import jax.numpy as jnp
from jax.experimental import pallas as pl
from jax.experimental.pallas import tpu as pltpu

_NB = 8


def _conv5x5(y16, w_ref, b_ref, pad_ref=None):
    n, H, W, C = y16.shape
    Cout = w_ref.shape[1] // 5
    Hp = H + 4
    if pad_ref is None:
        zh = jnp.zeros((n, 2, W, C), jnp.bfloat16)
        yh = jnp.concatenate([zh, y16, zh], axis=1)
        zw = jnp.zeros((n, Hp, 2, C), jnp.bfloat16)
        yp = jnp.concatenate([zw, yh, zw], axis=2)
    else:
        pad_ref[:, 0:2, :, :] = jnp.zeros((n, 2, W + 4, C), jnp.bfloat16)
        pad_ref[:, H + 2:Hp, :, :] = jnp.zeros((n, 2, W + 4, C),
                                               jnp.bfloat16)
        pad_ref[:, 2:H + 2, 0:2, :] = jnp.zeros((n, H, 2, C), jnp.bfloat16)
        pad_ref[:, 2:H + 2, W + 2:W + 4, :] = jnp.zeros((n, H, 2, C),
                                                        jnp.bfloat16)
        pad_ref[:, 2:H + 2, 2:W + 2, :] = y16
        yp = pad_ref[...]
    cols = jnp.concatenate(
        [yp[:, :, kw:kw + W, :] for kw in range(5)], axis=-1)
    p = jnp.dot(cols.reshape(n * Hp * W, 5 * C), w_ref[...],
                preferred_element_type=jnp.float32)
    p = p.reshape(n, Hp, W, 5 * Cout).astype(jnp.bfloat16)
    acc = (p[:, 0:H, :, 0:Cout].astype(jnp.float32)
           + p[:, 1:1 + H, :, Cout:2 * Cout])
    for kh in range(2, 5):
        acc = acc + p[:, kh:kh + H, :, kh * Cout:(kh + 1) * Cout]
    return jnp.maximum(acc + b_ref[...], 0.0)


def _pool2x2_16(y):
    n, H, W, C = y.shape
    return jnp.max(y.reshape(n, H // 2, 2, W // 2, 2, C),
                   axis=(2, 4)).astype(jnp.bfloat16)


def _convnet_kernel(x_ref, w1_ref, b1_ref, w2_ref, b2_ref, w3_ref, b3_ref,
                    w4_ref, b4_ref, w5_ref, b5_ref, w6_ref, b6_ref,
                    w7_ref, b7_ref, o_ref,
                    pad32_ref, pad32b_ref, pad16_ref, pad16b_ref,
                    pad8_ref, pad8b_ref):
    nb = x_ref.shape[0]
    x = x_ref[...]
    w1 = w1_ref[...]
    y = (x[..., 0:1] * w1[0:1, :] + x[..., 1:2] * w1[1:2, :]
         + x[..., 2:3] * w1[2:3, :] + b1_ref[...])
    y = jnp.maximum(y, 0.0).astype(jnp.bfloat16)

    y = _conv5x5(y, w2_ref, b2_ref, pad32_ref).astype(jnp.bfloat16)
    y = _conv5x5(y, w3_ref, b3_ref, pad32b_ref)
    p1 = _pool2x2_16(y)
    y = _conv5x5(p1, w4_ref, b4_ref, pad16_ref).astype(jnp.bfloat16)
    y = _conv5x5(y, w5_ref, b5_ref, pad16b_ref)
    p2 = _pool2x2_16(y)
    y = _conv5x5(p2, w6_ref, b6_ref, pad8_ref).astype(jnp.bfloat16)
    y = _conv5x5(y, w7_ref, b7_ref, pad8b_ref)
    p3 = _pool2x2_16(y)

    o_ref[0, :, 0:8192] = p1.reshape(nb, 8192)
    o_ref[0, :, 8192:12288] = p2.reshape(nb, 4096)
    o_ref[0, :, 12288:14336] = p3.reshape(nb, 2048)


def _fc1_kernel(x_ref, w1_ref, b1_ref, o_ref, acc_ref):
    k = pl.program_id(1)

    @pl.when(k == 0)
    def _():
        acc_ref[...] = jnp.zeros_like(acc_ref)

    acc_ref[...] += jnp.dot(x_ref[...], w1_ref[...].astype(jnp.bfloat16),
                            preferred_element_type=jnp.float32)

    @pl.when(k == pl.num_programs(1) - 1)
    def _():
        o_ref[...] = jnp.maximum(acc_ref[...] + b1_ref[...],
                                 0.0).astype(jnp.bfloat16)


def _head_kernel(h_ref, w2_ref, b2_ref, w3_ref, b3_ref, o_ref):
    h = jnp.maximum(jnp.dot(h_ref[...], w2_ref[...].astype(jnp.bfloat16),
                            preferred_element_type=jnp.float32)
                    + b2_ref[...], 0.0)
    logits = jnp.dot(h.astype(jnp.bfloat16),
                     w3_ref[...].astype(jnp.bfloat16),
                     preferred_element_type=jnp.float32) + b3_ref[...]
    m = jnp.max(logits, axis=-1, keepdims=True)
    e = jnp.exp(logits - m)
    o_ref[...] = e / jnp.sum(e, axis=-1, keepdims=True)


def kernel(conv1_w, conv1_b, conv2_w, conv2_b, conv3_w, conv3_b,
           conv4_w, conv4_b, conv5_w, conv5_b, conv6_w, conv6_b,
           conv7_w, conv7_b, fc1_w, fc1_b, fc2_w, fc2_b, fc3_w, fc3_b,
           x_nchw):
    N = x_nchw.shape[0]
    x = jnp.transpose(x_nchw, (0, 2, 3, 1))

    def wm(w):
        kh, kw, ci, co = w.shape
        return (w.transpose(1, 2, 0, 3).reshape(kw * ci, kh * co)
                .astype(jnp.bfloat16))

    conv_ws = [conv1_w.reshape(3, 3), conv1_b.reshape(1, 3),
               wm(conv2_w), conv2_b.reshape(1, 32),
               wm(conv3_w), conv3_b.reshape(1, 32),
               wm(conv4_w), conv4_b.reshape(1, 64),
               wm(conv5_w), conv5_b.reshape(1, 64),
               wm(conv6_w), conv6_b.reshape(1, 128),
               wm(conv7_w), conv7_b.reshape(1, 128)]

    def whole(w):
        r = len(w.shape)
        return pl.BlockSpec(w.shape, lambda n, _r=r: (0,) * _r)

    feats = pl.pallas_call(
        _convnet_kernel,
        out_shape=jax.ShapeDtypeStruct((N // _NB, _NB, 14336), jnp.bfloat16),
        grid=(N // _NB,),
        in_specs=[pl.BlockSpec((_NB, 32, 32, 3), lambda n: (n, 0, 0, 0))]
                 + [whole(w) for w in conv_ws],
        out_specs=pl.BlockSpec((1, _NB, 14336), lambda n: (n, 0, 0)),
        scratch_shapes=[
            pltpu.VMEM((_NB, 36, 36, 3), jnp.bfloat16),
            pltpu.VMEM((_NB, 36, 36, 32), jnp.bfloat16),
            pltpu.VMEM((_NB, 20, 20, 32), jnp.bfloat16),
            pltpu.VMEM((_NB, 20, 20, 64), jnp.bfloat16),
            pltpu.VMEM((_NB, 12, 12, 64), jnp.bfloat16),
            pltpu.VMEM((_NB, 12, 12, 128), jnp.bfloat16),
        ],
        compiler_params=pltpu.CompilerParams(
            dimension_semantics=("parallel",),
            vmem_limit_bytes=56 * 1024 * 1024),
    )(x, *conv_ws).reshape(N, 14336)

    TK = 2048
    NK = fc1_w.shape[0] // TK
    BN = 512
    h = pl.pallas_call(
        _fc1_kernel,
        out_shape=jax.ShapeDtypeStruct((N, 1024), jnp.bfloat16),
        grid=(2, NK),
        in_specs=[
            pl.BlockSpec((N, TK), lambda i, k: (0, k)),
            pl.BlockSpec((TK, BN), lambda i, k: (k, i)),
            pl.BlockSpec((1, BN), lambda i, k: (0, i)),
        ],
        out_specs=pl.BlockSpec((N, BN), lambda i, k: (0, i)),
        scratch_shapes=[pltpu.VMEM((N, BN), jnp.float32)],
        compiler_params=pltpu.CompilerParams(
            dimension_semantics=("parallel", "arbitrary"),
            vmem_limit_bytes=56 * 1024 * 1024),
    )(feats, fc1_w, fc1_b.reshape(1, 1024))

    BM = N // 2
    out = pl.pallas_call(
        _head_kernel,
        out_shape=jax.ShapeDtypeStruct((N, 43), jnp.float32),
        grid=(2,),
        in_specs=[
            pl.BlockSpec((BM, 1024), lambda i: (i, 0)),
            pl.BlockSpec((1024, 1024), lambda i: (0, 0)),
            pl.BlockSpec((1, 1024), lambda i: (0, 0)),
            pl.BlockSpec((1024, 43), lambda i: (0, 0)),
            pl.BlockSpec((1, 43), lambda i: (0, 0)),
        ],
        out_specs=pl.BlockSpec((BM, 43), lambda i: (i, 0)),
        compiler_params=pltpu.CompilerParams(
            dimension_semantics=("parallel",),
            vmem_limit_bytes=56 * 1024 * 1024),
    )(h, fc2_w, fc2_b.reshape(1, 1024), fc3_w, fc3_b.reshape(1, 43))
    return out


# speedup vs baseline: 1.0581x; 1.0581x over previous
import jax
import jax.numpy as jnp
from jax.experimental import pallas as pl
from jax.experimental.pallas import tpu as pltpu

_NB = 8


def _conv5x5(y16, w_ref, b_ref):
    n, H, W, C = y16.shape
    Cout = w_ref.shape[1] // 5
    Hp = H + 4
    zh = jnp.zeros((n, 2, W, C), jnp.bfloat16)
    yh = jnp.concatenate([zh, y16, zh], axis=1)
    zw = jnp.zeros((n, Hp, 2, C), jnp.bfloat16)
    yp = jnp.concatenate([zw, yh, zw], axis=2)
    cols = jnp.concatenate(
        [yp[:, :, kw:kw + W, :] for kw in range(5)], axis=-1)
    p = jnp.dot(cols.reshape(n * Hp * W, 5 * C), w_ref[...],
                preferred_element_type=jnp.float32)
    p = p.reshape(n, Hp, W, 5 * Cout).astype(jnp.bfloat16)
    acc = (p[:, 0:H, :, 0:Cout].astype(jnp.float32)
           + p[:, 1:1 + H, :, Cout:2 * Cout])
    for kh in range(2, 5):
        acc = acc + p[:, kh:kh + H, :, kh * Cout:(kh + 1) * Cout]
    return jnp.maximum(acc + b_ref[...], 0.0)


def _pool2x2_16(y):
    n, H, W, C = y.shape
    return jnp.max(y.reshape(n, H // 2, 2, W // 2, 2, C),
                   axis=(2, 4)).astype(jnp.bfloat16)


def _convnet_kernel(x_ref, w1_ref, b1_ref, w2_ref, b2_ref, w3_ref, b3_ref,
                    w4_ref, b4_ref, w5_ref, b5_ref, w6_ref, b6_ref,
                    w7_ref, b7_ref, o_ref):
    nb = x_ref.shape[0]
    x = x_ref[...]
    w1 = w1_ref[...]
    y = (x[..., 0:1] * w1[0:1, :] + x[..., 1:2] * w1[1:2, :]
         + x[..., 2:3] * w1[2:3, :] + b1_ref[...])
    y = jnp.maximum(y, 0.0).astype(jnp.bfloat16)

    y = _conv5x5(y, w2_ref, b2_ref).astype(jnp.bfloat16)
    y = _conv5x5(y, w3_ref, b3_ref)
    p1 = _pool2x2_16(y)
    y = _conv5x5(p1, w4_ref, b4_ref).astype(jnp.bfloat16)
    y = _conv5x5(y, w5_ref, b5_ref)
    p2 = _pool2x2_16(y)
    y = _conv5x5(p2, w6_ref, b6_ref).astype(jnp.bfloat16)
    y = _conv5x5(y, w7_ref, b7_ref)
    p3 = _pool2x2_16(y)

    o_ref[0, :, 0:8192] = p1.reshape(nb, 8192)
    o_ref[0, :, 8192:12288] = p2.reshape(nb, 4096)
    o_ref[0, :, 12288:14336] = p3.reshape(nb, 2048)


def _fc1_kernel(x_ref, w1_ref, b1_ref, o_ref, acc_ref):
    k = pl.program_id(1)

    @pl.when(k == 0)
    def _():
        acc_ref[...] = jnp.zeros_like(acc_ref)

    acc_ref[...] += jnp.dot(x_ref[...], w1_ref[...].astype(jnp.bfloat16),
                            preferred_element_type=jnp.float32)

    @pl.when(k == pl.num_programs(1) - 1)
    def _():
        o_ref[...] = jnp.maximum(acc_ref[...] + b1_ref[...],
                                 0.0).astype(jnp.bfloat16)


def _head_kernel(h_ref, w2_ref, b2_ref, w3_ref, b3_ref, o_ref):
    h = jnp.maximum(jnp.dot(h_ref[...], w2_ref[...].astype(jnp.bfloat16),
                            preferred_element_type=jnp.float32)
                    + b2_ref[...], 0.0)
    logits = jnp.dot(h.astype(jnp.bfloat16),
                     w3_ref[...].astype(jnp.bfloat16),
                     preferred_element_type=jnp.float32) + b3_ref[...]
    m = jnp.max(logits, axis=-1, keepdims=True)
    e = jnp.exp(logits - m)
    o_ref[...] = e / jnp.sum(e, axis=-1, keepdims=True)


def kernel(conv1_w, conv1_b, conv2_w, conv2_b, conv3_w, conv3_b,
           conv4_w, conv4_b, conv5_w, conv5_b, conv6_w, conv6_b,
           conv7_w, conv7_b, fc1_w, fc1_b, fc2_w, fc2_b, fc3_w, fc3_b,
           x_nchw):
    N = x_nchw.shape[0]
    x = jnp.transpose(x_nchw, (0, 2, 3, 1))

    def wm(w):
        kh, kw, ci, co = w.shape
        return (w.transpose(1, 2, 0, 3).reshape(kw * ci, kh * co)
                .astype(jnp.bfloat16))

    conv_ws = [conv1_w.reshape(3, 3), conv1_b.reshape(1, 3),
               wm(conv2_w), conv2_b.reshape(1, 32),
               wm(conv3_w), conv3_b.reshape(1, 32),
               wm(conv4_w), conv4_b.reshape(1, 64),
               wm(conv5_w), conv5_b.reshape(1, 64),
               wm(conv6_w), conv6_b.reshape(1, 128),
               wm(conv7_w), conv7_b.reshape(1, 128)]

    def whole(w):
        r = len(w.shape)
        return pl.BlockSpec(w.shape, lambda n, _r=r: (0,) * _r)

    feats = pl.pallas_call(
        _convnet_kernel,
        out_shape=jax.ShapeDtypeStruct((N // _NB, _NB, 14336), jnp.bfloat16),
        grid=(N // _NB,),
        in_specs=[pl.BlockSpec((_NB, 32, 32, 3), lambda n: (n, 0, 0, 0))]
                 + [whole(w) for w in conv_ws],
        out_specs=pl.BlockSpec((1, _NB, 14336), lambda n: (n, 0, 0)),
        compiler_params=pltpu.CompilerParams(
            dimension_semantics=("parallel",),
            vmem_limit_bytes=56 * 1024 * 1024),
    )(x, *conv_ws).reshape(N, 14336)

    TK = 2048
    NK = fc1_w.shape[0] // TK
    BN = 512
    h = pl.pallas_call(
        _fc1_kernel,
        out_shape=jax.ShapeDtypeStruct((N, 1024), jnp.bfloat16),
        grid=(2, NK),
        in_specs=[
            pl.BlockSpec((N, TK), lambda i, k: (0, k)),
            pl.BlockSpec((TK, BN), lambda i, k: (k, i)),
            pl.BlockSpec((1, BN), lambda i, k: (0, i)),
        ],
        out_specs=pl.BlockSpec((N, BN), lambda i, k: (0, i)),
        scratch_shapes=[pltpu.VMEM((N, BN), jnp.float32)],
        compiler_params=pltpu.CompilerParams(
            dimension_semantics=("parallel", "arbitrary"),
            vmem_limit_bytes=56 * 1024 * 1024),
    )(feats, fc1_w, fc1_b.reshape(1, 1024))

    BM = N // 2
    out = pl.pallas_call(
        _head_kernel,
        out_shape=jax.ShapeDtypeStruct((N, 43), jnp.float32),
        grid=(2,),
        in_specs=[
            pl.BlockSpec((BM, 1024), lambda i: (i, 0)),
            pl.BlockSpec((1024, 1024), lambda i: (0, 0)),
            pl.BlockSpec((1, 1024), lambda i: (0, 0)),
            pl.BlockSpec((1024, 43), lambda i: (0, 0)),
            pl.BlockSpec((1, 43), lambda i: (0, 0)),
        ],
        out_specs=pl.BlockSpec((BM, 43), lambda i: (i, 0)),
        compiler_params=pltpu.CompilerParams(
            dimension_semantics=("parallel",),
            vmem_limit_bytes=56 * 1024 * 1024),
    )(h, fc2_w, fc2_b.reshape(1, 1024), fc3_w, fc3_b.reshape(1, 43))
    return out
